# baseline (speedup 1.0000x reference)
import jax
import jax.numpy as jnp
from jax import lax
from jax.experimental import pallas as pl
from jax.experimental.pallas import tpu as pltpu
from jax.experimental.pallas import tpu_sc as plsc

_NC, _NS, _L = 2, 16, 16
_NW = _NC * _NS
_B = 16384
_BPW = _B // _NW
_D = 32
_G = _D // 8
_GROUPS = _BPW // _L
_BIAS = 3.5


def _sc_body(user_h, item_h, label_h, utab_h, itab_h,
             pred_h, rmse_h, part_h,
             idxu, idxi, stg_u, stg_v, fmaj_u, fmaj_v,
             labv, predv, rmsev, sqv, sem):
    wid = lax.axis_index("s") * _NC + lax.axis_index("c")
    base = wid * _BPW

    pltpu.sync_copy(user_h.at[pl.ds(base, _BPW)], idxu)
    pltpu.sync_copy(item_h.at[pl.ds(base, _BPW)], idxi)
    pltpu.sync_copy(label_h.at[pl.ds(base, _BPW)], labv)

    lane = lax.iota(jnp.int32, _L)

    def k_body(k, _):
        o = pl.multiple_of(k * _L, _L)
        ru = idxu[pl.ds(o, _L)]
        ri = idxi[pl.ds(o, _L)]
        cu = jnp.bitwise_and(ru, 127)
        ci = jnp.bitwise_and(ri, 127)
        for g in range(_G):
            cps = []
            for i in range(_L):
                wu = lax.shift_right_logical(ru[i], 7) * 128
                wi = lax.shift_right_logical(ri[i], 7) * 128
                cps.append(pltpu.async_copy(
                    utab_h.at[pl.ds(8 * g, 8), pl.ds(wu, 128)],
                    stg_u.at[i], sem))
                cps.append(pltpu.async_copy(
                    itab_h.at[pl.ds(8 * g, 8), pl.ds(wi, 128)],
                    stg_v.at[i], sem))
            for c in cps:
                c.wait()
            for f in range(8):
                fs = jnp.full((_L,), f, jnp.int32)
                uf = plsc.load_gather(stg_u, [lane, fs, cu])
                vf = plsc.load_gather(stg_v, [lane, fs, ci])
                fmaj_u[8 * g + f, pl.ds(o, _L)] = uf
                fmaj_v[8 * g + f, pl.ds(o, _L)] = vf
        return 0

    lax.fori_loop(0, _GROUPS, k_body, 0)

    def g_body(g, sq_acc):
        o = pl.multiple_of(g * _L, _L)
        acc = jnp.zeros((_L,), jnp.float32)
        for d in range(_D):
            acc = acc + fmaj_u[d, pl.ds(o, _L)] * fmaj_v[d, pl.ds(o, _L)]
        pred16 = acc + _BIAS
        predv[pl.ds(o, _L)] = pred16
        diff = pred16 - labv[pl.ds(o, _L)]
        rmsev[pl.ds(o, _L)] = jnp.abs(diff)
        return sq_acc + diff * diff

    sq = lax.fori_loop(0, _GROUPS, g_body, jnp.zeros((_L,), jnp.float32))
    sqv[...] = sq

    pltpu.sync_copy(predv, pred_h.at[pl.ds(base, _BPW)])
    pltpu.sync_copy(rmsev, rmse_h.at[pl.ds(base, _BPW)])
    pltpu.sync_copy(sqv, part_h.at[pl.ds(wid * _L, _L)])


def _obj_body(p_ref, o_ref):
    o_ref[0, 0] = jnp.sum(p_ref[...]) * (1.0 / _B)


def kernel(user, item, label, user_table, item_table):
    f32 = jnp.float32
    sc_fn = pl.kernel(
        _sc_body,
        out_type=(
            jax.ShapeDtypeStruct((_B,), f32),
            jax.ShapeDtypeStruct((_B,), f32),
            jax.ShapeDtypeStruct((_NW * _L,), f32),
        ),
        mesh=plsc.VectorSubcoreMesh(core_axis_name="c", subcore_axis_name="s"),
        compiler_params=pltpu.CompilerParams(needs_layout_passes=False),
        scratch_types=[
            pltpu.VMEM((_BPW,), jnp.int32),
            pltpu.VMEM((_BPW,), jnp.int32),
            pltpu.VMEM((_L, 8, 128), f32),
            pltpu.VMEM((_L, 8, 128), f32),
            pltpu.VMEM((_D, _BPW), f32),
            pltpu.VMEM((_D, _BPW), f32),
            pltpu.VMEM((_BPW,), f32),
            pltpu.VMEM((_BPW,), f32),
            pltpu.VMEM((_BPW,), f32),
            pltpu.VMEM((_L,), f32),
            pltpu.SemaphoreType.DMA,
        ],
    )
    pred, rmse, part = sc_fn(user, item, label, user_table.T, item_table.T)

    obj2 = pl.pallas_call(
        _obj_body,
        out_shape=jax.ShapeDtypeStruct((1, 1), f32),
        out_specs=pl.BlockSpec(memory_space=pltpu.SMEM),
    )(part.reshape(_NW, _L))

    return (pred, obj2[0, 0], rmse)
